# speedup vs baseline: 3.4585x; 3.4585x over previous
import jax
import jax.numpy as jnp
from jax import lax
from jax.experimental import pallas as pl
from jax.experimental.pallas import tpu as pltpu
from jax.experimental.pallas import tpu_sc as plsc

NUM_ROWS = 100000
DIM = 128

B_TOTAL = 16384 * 50
CHUNK = 128
NUM_CHUNKS = B_TOTAL // CHUNK
NC, NS = 2, 16
NW = NC * NS
CHUNKS_PER_W = NUM_CHUNKS // NW
NBUF = 4
GROUPS = CHUNKS_PER_W // NBUF


def _body(idx_hbm, table_hbm, out_hbm, idx_v, bufs, *sems):
    gsem = sems[:NBUF]
    osem = sems[NBUF:]
    wid = lax.axis_index("s") * NC + lax.axis_index("c")
    row_base = wid * CHUNKS_PER_W

    pltpu.sync_copy(idx_hbm.at[pl.ds(row_base, CHUNKS_PER_W)], idx_v)

    out_base = row_base * CHUNK

    def start_gather(j, b):
        pltpu.async_copy(table_hbm.at[idx_v.at[j]], bufs.at[b], gsem[b])

    def wait_gather(j, b):
        pltpu.make_async_copy(table_hbm.at[idx_v.at[j]], bufs.at[b], gsem[b]).wait()

    def start_out(j, b):
        pltpu.async_copy(bufs.at[b], out_hbm.at[pl.ds(out_base + j * CHUNK, CHUNK)], osem[b])

    def wait_out(j, b):
        pltpu.make_async_copy(bufs.at[b], out_hbm.at[pl.ds(out_base + j * CHUNK, CHUNK)], osem[b]).wait()

    for b in range(NBUF):
        start_gather(b, b)

    @pl.loop(0, GROUPS - 1)
    def _(g):
        for b in range(NBUF):
            j = g * NBUF + b
            wait_gather(j, b)
            start_out(j, b)
            wait_out(j, b)
            start_gather(j + NBUF, b)

    for b in range(NBUF):
        j = (GROUPS - 1) * NBUF + b
        wait_gather(j, b)
        start_out(j, b)
        wait_out(j, b)


def _make_kernel():
    mesh = plsc.VectorSubcoreMesh(core_axis_name="c", subcore_axis_name="s")
    return pl.kernel(
        _body,
        out_type=jax.ShapeDtypeStruct((B_TOTAL, DIM), jnp.float32),
        mesh=mesh,
        scratch_types=[
            pltpu.VMEM((CHUNKS_PER_W, CHUNK), jnp.int32),
            pltpu.VMEM((NBUF, CHUNK, DIM), jnp.float32),
        ] + [pltpu.SemaphoreType.DMA] * (2 * NBUF),
    )


def kernel(input_, weight):
    idx = input_.astype(jnp.int32).reshape(NUM_CHUNKS, CHUNK)
    out = _make_kernel()(idx, weight)
    return out.reshape(input_.shape[0], input_.shape[1], DIM)


# speedup vs baseline: 3.4624x; 1.0011x over previous
import jax
import jax.numpy as jnp
from jax import lax
from jax.experimental import pallas as pl
from jax.experimental.pallas import tpu as pltpu
from jax.experimental.pallas import tpu_sc as plsc

NUM_ROWS = 100000
DIM = 128

B_TOTAL = 16384 * 50
CHUNK = 128
NUM_CHUNKS = B_TOTAL // CHUNK
NC, NS = 2, 16
NW = NC * NS
CHUNKS_PER_W = NUM_CHUNKS // NW
NBUF = 4
GROUPS = CHUNKS_PER_W // NBUF


def _body(idx_hbm, table_hbm, out_hbm, idx_v, bufs, *sems):
    gsem = sems[:NBUF]
    osem = sems[NBUF:]
    wid = lax.axis_index("s") * NC + lax.axis_index("c")
    row_base = wid * CHUNKS_PER_W

    pltpu.sync_copy(idx_hbm.at[pl.ds(row_base, CHUNKS_PER_W)], idx_v)

    out_base = row_base * CHUNK

    def start_gather(j, b):
        pltpu.async_copy(table_hbm.at[idx_v.at[j]], bufs.at[b], gsem[b])

    def wait_gather(j, b):
        pltpu.make_async_copy(table_hbm.at[idx_v.at[j]], bufs.at[b], gsem[b]).wait()

    def start_out(j, b):
        pltpu.async_copy(bufs.at[b], out_hbm.at[pl.ds(out_base + j * CHUNK, CHUNK)], osem[b])

    def wait_out(j, b):
        pltpu.make_async_copy(bufs.at[b], out_hbm.at[pl.ds(out_base + j * CHUNK, CHUNK)], osem[b]).wait()

    for b in range(NBUF):
        start_gather(b, b)

    @pl.loop(0, GROUPS - 1)
    def _(g):
        for b in range(NBUF):
            j = g * NBUF + b
            wait_gather(j, b)
            start_out(j, b)
        for b in range(NBUF):
            j = g * NBUF + b
            wait_out(j, b)
            start_gather(j + NBUF, b)

    for b in range(NBUF):
        j = (GROUPS - 1) * NBUF + b
        wait_gather(j, b)
        start_out(j, b)
    for b in range(NBUF):
        j = (GROUPS - 1) * NBUF + b
        wait_out(j, b)


def _make_kernel():
    mesh = plsc.VectorSubcoreMesh(core_axis_name="c", subcore_axis_name="s")
    return pl.kernel(
        _body,
        out_type=jax.ShapeDtypeStruct((B_TOTAL, DIM), jnp.float32),
        mesh=mesh,
        scratch_types=[
            pltpu.VMEM((CHUNKS_PER_W, CHUNK), jnp.int32),
            pltpu.VMEM((NBUF, CHUNK, DIM), jnp.float32),
        ] + [pltpu.SemaphoreType.DMA] * (2 * NBUF),
    )


def kernel(input_, weight):
    idx = input_.astype(jnp.int32).reshape(NUM_CHUNKS, CHUNK)
    out = _make_kernel()(idx, weight)
    return out.reshape(input_.shape[0], input_.shape[1], DIM)


# speedup vs baseline: 6.3036x; 1.8206x over previous
import jax
import jax.numpy as jnp
from jax import lax
from jax.experimental import pallas as pl
from jax.experimental.pallas import tpu as pltpu
from jax.experimental.pallas import tpu_sc as plsc

NUM_ROWS = 100000
DIM = 128
BATCH = 16384
SEQ = 50

ROWS_PER_CHUNK = 2
CHUNK = ROWS_PER_CHUNK * SEQ
NUM_CHUNKS = BATCH // ROWS_PER_CHUNK
NC, NS = 2, 16
NW = NC * NS
CHUNKS_PER_W = NUM_CHUNKS // NW
NBUF = 4
GROUPS = CHUNKS_PER_W // NBUF


def _body(idx_hbm, table_hbm, out_hbm, idx_v, bufs, *sems):
    gsem = sems[:NBUF]
    osem = sems[NBUF:]
    wid = lax.axis_index("s") * NC + lax.axis_index("c")
    chunk_base = wid * CHUNKS_PER_W

    pltpu.sync_copy(idx_hbm.at[pl.ds(chunk_base, CHUNKS_PER_W)], idx_v)

    row_base = chunk_base * ROWS_PER_CHUNK

    def start_gather(j, b):
        pltpu.async_copy(table_hbm.at[idx_v.at[j]], bufs.at[b], gsem[b])

    def wait_gather(j, b):
        pltpu.make_async_copy(table_hbm.at[idx_v.at[j]], bufs.at[b], gsem[b]).wait()

    def start_out(j, b):
        row = row_base + j * ROWS_PER_CHUNK
        for r in range(ROWS_PER_CHUNK):
            pltpu.async_copy(
                bufs.at[b].at[pl.ds(r * SEQ, SEQ)], out_hbm.at[row + r], osem[b])

    def wait_out(j, b):
        row = row_base + j * ROWS_PER_CHUNK
        for r in range(ROWS_PER_CHUNK):
            pltpu.make_async_copy(
                bufs.at[b].at[pl.ds(r * SEQ, SEQ)], out_hbm.at[row + r], osem[b]).wait()

    for b in range(NBUF):
        start_gather(b, b)

    @pl.loop(0, GROUPS - 1)
    def _(g):
        for b in range(NBUF):
            j = g * NBUF + b
            wait_gather(j, b)
            start_out(j, b)
        for b in range(NBUF):
            j = g * NBUF + b
            wait_out(j, b)
            start_gather(j + NBUF, b)

    for b in range(NBUF):
        j = (GROUPS - 1) * NBUF + b
        wait_gather(j, b)
        start_out(j, b)
    for b in range(NBUF):
        j = (GROUPS - 1) * NBUF + b
        wait_out(j, b)


def _make_kernel():
    mesh = plsc.VectorSubcoreMesh(core_axis_name="c", subcore_axis_name="s")
    return pl.kernel(
        _body,
        out_type=jax.ShapeDtypeStruct((BATCH, SEQ, DIM), jnp.float32),
        mesh=mesh,
        scratch_types=[
            pltpu.VMEM((CHUNKS_PER_W, CHUNK), jnp.int32),
            pltpu.VMEM((NBUF, CHUNK, DIM), jnp.float32),
        ] + [pltpu.SemaphoreType.DMA] * (2 * NBUF),
    )


def kernel(input_, weight):
    idx = input_.astype(jnp.int32).reshape(NUM_CHUNKS, CHUNK)
    return _make_kernel()(idx, weight)


# speedup vs baseline: 6.3304x; 1.0043x over previous
import jax
import jax.numpy as jnp
from jax import lax
from jax.experimental import pallas as pl
from jax.experimental.pallas import tpu as pltpu
from jax.experimental.pallas import tpu_sc as plsc

NUM_ROWS = 100000
DIM = 128
BATCH = 16384
SEQ = 50

GATHER_ROWS = 100
KB = 4
GPC = KB * SEQ // GATHER_ROWS
NC, NS = 2, 16
NW = NC * NS
ROWS_PER_W = BATCH // NW
HALVES = 2
ROWS_PER_H = ROWS_PER_W // HALVES
IDX_PER_H = ROWS_PER_H * SEQ // GATHER_ROWS
CHUNKS_PER_H = ROWS_PER_H // KB
NBUF = 4
GROUPS = CHUNKS_PER_H // NBUF


def _body(idx_hbm, table_hbm, out_hbm, idx_v, bufs, *sems):
    gsem = sems[:NBUF]
    osem = sems[NBUF:]
    wid = lax.axis_index("s") * NC + lax.axis_index("c")
    row_base = wid * ROWS_PER_W
    idx_base = wid * (HALVES * IDX_PER_H)

    def start_gather(j, b):
        for k in range(GPC):
            pltpu.async_copy(
                table_hbm.at[idx_v.at[GPC * j + k]],
                bufs.at[b].at[pl.ds(k * GATHER_ROWS, GATHER_ROWS)], gsem[b])

    def wait_gather(j, b):
        for k in range(GPC):
            pltpu.make_async_copy(
                table_hbm.at[idx_v.at[GPC * j + k]],
                bufs.at[b].at[pl.ds(k * GATHER_ROWS, GATHER_ROWS)], gsem[b]).wait()

    def make_out(h, j, b):
        row = row_base + h * ROWS_PER_H + j * KB
        return pltpu.make_async_copy(
            bufs.at[b].reshape(KB, SEQ, DIM), out_hbm.at[pl.ds(row, KB)], osem[b])

    for h in range(HALVES):
        pltpu.sync_copy(
            idx_hbm.at[pl.ds(idx_base + h * IDX_PER_H, IDX_PER_H)], idx_v)

        for b in range(NBUF):
            start_gather(b, b)

        @pl.loop(0, GROUPS - 1)
        def _(g):
            for b in range(NBUF):
                j = g * NBUF + b
                wait_gather(j, b)
                make_out(h, j, b).start()
            for b in range(NBUF):
                j = g * NBUF + b
                make_out(h, j, b).wait()
                start_gather(j + NBUF, b)

        for b in range(NBUF):
            j = (GROUPS - 1) * NBUF + b
            wait_gather(j, b)
            make_out(h, j, b).start()
        for b in range(NBUF):
            j = (GROUPS - 1) * NBUF + b
            make_out(h, j, b).wait()


def _make_kernel():
    mesh = plsc.VectorSubcoreMesh(core_axis_name="c", subcore_axis_name="s")
    return pl.kernel(
        _body,
        out_type=jax.ShapeDtypeStruct((BATCH, SEQ, DIM), jnp.float32),
        mesh=mesh,
        scratch_types=[
            pltpu.VMEM((IDX_PER_H, GATHER_ROWS), jnp.int32),
            pltpu.VMEM((NBUF, KB * SEQ, DIM), jnp.float32),
        ] + [pltpu.SemaphoreType.DMA] * (2 * NBUF),
    )


def kernel(input_, weight):
    idx = input_.astype(jnp.int32).reshape(BATCH * SEQ // GATHER_ROWS, GATHER_ROWS)
    return _make_kernel()(idx, weight)
